# Initial kernel scaffold; baseline (speedup 1.0000x reference)
import jax
import jax.numpy as jnp
from jax.experimental import pallas as pl


def kernel(p, x, o, W, b, gamma, beta):
    raise NotImplementedError("write your pallas kernel here")


# speedup vs baseline: 1.1856x; 1.1856x over previous
import functools

import jax
import jax.numpy as jnp
from jax import lax
from jax.experimental import pallas as pl
from jax.experimental.pallas import tpu as pltpu
from jax.experimental.pallas import tpu_sc as plsc

N = 10000
C = 128
OUT = 256
STRIDE = 4
NSAMPLE = 32

M = N // STRIDE
MP = 2560
NP = 10240
BM = 128
TCOL = 512
NT = NP // TCOL
F = C + 3
FP = 144

NEG = -3.0e38
POS = 3.0e38
IBIG = 2 ** 30


def _topk_body(np_ref, pt_ref, idx_ref, cur_ref):
    a0 = np_ref[:, 0:1]
    a1 = np_ref[:, 1:2]
    a2 = np_ref[:, 2:3]
    pn = a0 * a0 + a1 * a1 + a2 * a2

    def fill(t, _):
        sl = pl.ds(t * TCOL, TCOL)
        p0 = pt_ref[0:1, sl]
        p1 = pt_ref[1:2, sl]
        p2 = pt_ref[2:3, sl]
        pp = p0 * p0 + p1 * p1 + p2 * p2
        mm = jax.lax.dot_general(
            np_ref[:, 0:3], pt_ref[0:3, sl],
            (((1,), (0,)), ((), ())))
        cur_ref[:, sl] = pn + pp - 2.0 * mm
        return 0

    lax.fori_loop(0, NT, fill, 0)

    lv = jnp.full((BM, 1), NEG, dtype=jnp.float32)
    li = jnp.full((BM, 1), -1, dtype=jnp.int32)
    for k in range(NSAMPLE):
        def scan_tile(t, carry):
            bv, bi = carry
            sl = pl.ds(t * TCOL, TCOL)
            v = cur_ref[:, sl]
            gi = (lax.broadcasted_iota(jnp.int32, (BM, TCOL), 1)
                  + t * TCOL)
            elig = (v > lv) | ((v == lv) & (gi > li))
            vm = jnp.where(elig, v, POS)
            tmin = jnp.min(vm, axis=1, keepdims=True)
            targ = jnp.min(jnp.where(vm == tmin, gi, IBIG),
                           axis=1, keepdims=True)
            take = (tmin < bv) | ((tmin == bv) & (targ < bi))
            return (jnp.where(take, tmin, bv), jnp.where(take, targ, bi))

        bv, bi = lax.fori_loop(
            0, NT, scan_tile,
            (jnp.full((BM, 1), POS, dtype=jnp.float32),
             jnp.full((BM, 1), IBIG, dtype=jnp.int32)))
        idx_ref[:, k:k + 1] = bi
        lv, li = bv, bi


def _run_topk(np_blk, pt_pad):
    return pl.pallas_call(
        _topk_body,
        grid=(MP // BM,),
        in_specs=[
            pl.BlockSpec((BM, 128), lambda i: (i, 0)),
            pl.BlockSpec((8, NP), lambda i: (0, 0)),
        ],
        out_specs=pl.BlockSpec((BM, NSAMPLE), lambda i: (i, 0)),
        out_shape=jax.ShapeDtypeStruct((MP, NSAMPLE), jnp.int32),
        scratch_shapes=[pltpu.VMEM((BM, NP), jnp.float32)],
    )(np_blk, pt_pad)


CH = 4
ROWS_W = MP // 32
NCHUNK = ROWS_W // CH


def _pool_body(xp_hbm, idx_hbm, out_hbm, idx_v, rows_v, pool_v, sem):
    cid = lax.axis_index("c")
    sid = lax.axis_index("s")
    wid = sid * 2 + cid
    base = wid * ROWS_W

    def chunk(ci, _):
        r0 = base + ci * CH
        pltpu.sync_copy(idx_hbm.at[pl.ds(r0 * NSAMPLE, CH * NSAMPLE)], idx_v)
        pltpu.async_copy(xp_hbm.at[idx_v], rows_v, sem).wait()

        def over_r(rr, _):
            def over_s(s, accs):
                row = rr * NSAMPLE + s
                return tuple(
                    jnp.maximum(accs[cc], rows_v[row, pl.ds(cc * 16, 16)])
                    for cc in range(FP // 16))

            accs0 = tuple(rows_v[rr * NSAMPLE, pl.ds(cc * 16, 16)]
                          for cc in range(FP // 16))
            accs = lax.fori_loop(1, NSAMPLE, over_s, accs0)
            prow = ci * CH + rr
            for cc in range(FP // 16):
                pool_v[prow, pl.ds(cc * 16, 16)] = accs[cc]
            return 0

        lax.fori_loop(0, CH, over_r, 0)
        return 0

    lax.fori_loop(0, NCHUNK, chunk, 0)
    pltpu.sync_copy(pool_v, out_hbm.at[pl.ds(base, ROWS_W)])


def _run_pool(xp, idx_flat):
    mesh = plsc.VectorSubcoreMesh(core_axis_name="c", subcore_axis_name="s")
    fn = functools.partial(
        pl.kernel,
        mesh=mesh,
        compiler_params=pltpu.CompilerParams(use_tc_tiling_on_sc=False),
        out_type=jax.ShapeDtypeStruct((MP, FP), jnp.float32),
        scratch_types=[
            pltpu.VMEM((CH * NSAMPLE,), jnp.int32),
            pltpu.VMEM((CH * NSAMPLE, FP), jnp.float32),
            pltpu.VMEM((ROWS_W, FP), jnp.float32),
            pltpu.SemaphoreType.DMA,
        ],
    )(_pool_body)
    return fn(xp, idx_flat)


RT = 256
NRT = MP // RT


def _dense_body(pool_ref, npad_ref, w_ref, sgb_ref, out_ref, h_ref):
    w = w_ref[:, :]
    bvec = sgb_ref[0:1, :]
    gamma = sgb_ref[1:2, :]
    beta = sgb_ref[2:3, :]

    def mm(t, _):
        sl = pl.ds(t * RT, RT)
        adj = pool_ref[sl, :] - npad_ref[sl, :]
        h_ref[sl, :] = jax.lax.dot(adj, w) + bvec
        return 0

    lax.fori_loop(0, NRT, mm, 0)

    def sum_tile(t, acc):
        sl = pl.ds(t * RT, RT)
        gi = lax.broadcasted_iota(jnp.int32, (RT, 1), 0) + t * RT
        msk = gi < M
        return acc + jnp.sum(jnp.where(msk, h_ref[sl, :], 0.0),
                             axis=0, keepdims=True)

    mean = lax.fori_loop(0, NRT, sum_tile,
                         jnp.zeros((1, OUT), jnp.float32)) / M

    def var_tile(t, acc):
        sl = pl.ds(t * RT, RT)
        gi = lax.broadcasted_iota(jnp.int32, (RT, 1), 0) + t * RT
        msk = gi < M
        d = h_ref[sl, :] - mean
        return acc + jnp.sum(jnp.where(msk, d * d, 0.0),
                             axis=0, keepdims=True)

    var = lax.fori_loop(0, NRT, var_tile,
                        jnp.zeros((1, OUT), jnp.float32)) / M
    inv = lax.rsqrt(var + 1e-5)

    def norm_tile(t, _):
        sl = pl.ds(t * RT, RT)
        d = h_ref[sl, :] - mean
        out_ref[sl, :] = jnp.maximum(d * inv * gamma + beta, 0.0)
        return 0

    lax.fori_loop(0, NRT, norm_tile, 0)


def _run_dense(pooled, npad, w_pad, sgb):
    return pl.pallas_call(
        _dense_body,
        out_shape=jax.ShapeDtypeStruct((MP, OUT), jnp.float32),
        scratch_shapes=[pltpu.VMEM((MP, OUT), jnp.float32)],
    )(pooled, npad, w_pad, sgb)


def kernel(p, x, o, W, b, gamma, beta):
    n_p = p[::STRIDE]
    n_o = o // STRIDE

    np_blk = jnp.zeros((MP, 128), jnp.float32).at[:M, :3].set(n_p)
    pt_pad = jnp.full((8, NP), 0.0, jnp.float32)
    pt_pad = pt_pad.at[:3, :N].set(p.T)
    pt_pad = pt_pad.at[:3, N:].set(1e18)
    nn_idx = _run_topk(np_blk, pt_pad)

    xp = jnp.zeros((N, FP), jnp.float32)
    xp = xp.at[:, :3].set(p).at[:, 3:3 + C].set(x)
    pooled = _run_pool(xp, nn_idx.reshape(-1))

    npad = jnp.zeros((MP, FP), jnp.float32).at[:M, :3].set(n_p)
    w_pad = jnp.zeros((FP, OUT), jnp.float32).at[:F, :].set(W)
    sgb = jnp.zeros((8, OUT), jnp.float32)
    sgb = sgb.at[0].set(b).at[1].set(gamma).at[2].set(beta)
    x_out = _run_dense(pooled, npad, w_pad, sgb)[:M]

    return (n_p, x_out, n_o)


# speedup vs baseline: 2.5362x; 2.1392x over previous
import functools

import jax
import jax.numpy as jnp
from jax import lax
from jax.experimental import pallas as pl
from jax.experimental.pallas import tpu as pltpu
from jax.experimental.pallas import tpu_sc as plsc

N = 10000
C = 128
OUT = 256
STRIDE = 4
NSAMPLE = 32

M = N // STRIDE
MP = 2560
NP = 10240
BM = 256
TCOL = 1024
NT = NP // TCOL
F = C + 3
FP = 144

NEG = -3.0e38
POS = 3.0e38
IBIG = 2 ** 30


def _topk_body(np_ref, pt_ref, idx_ref, cur_ref):
    a0 = np_ref[:, 0:1]
    a1 = np_ref[:, 1:2]
    a2 = np_ref[:, 2:3]
    pn = a0 * a0 + a1 * a1 + a2 * a2

    def fill(t, _):
        sl = pl.ds(t * TCOL, TCOL)
        p0 = pt_ref[0:1, sl]
        p1 = pt_ref[1:2, sl]
        p2 = pt_ref[2:3, sl]
        pp = p0 * p0 + p1 * p1 + p2 * p2
        mm = jax.lax.dot_general(
            np_ref[:, 0:3], pt_ref[0:3, sl],
            (((1,), (0,)), ((), ())))
        cur_ref[:, sl] = pn + pp - 2.0 * mm
        return 0

    lax.fori_loop(0, NT, fill, 0)

    lv = jnp.full((BM, 1), NEG, dtype=jnp.float32)
    li = jnp.full((BM, 1), -1, dtype=jnp.int32)
    for k in range(NSAMPLE):
        def scan_tile(t, carry):
            bv, bi = carry
            sl = pl.ds(t * TCOL, TCOL)
            v = cur_ref[:, sl]
            gi = (lax.broadcasted_iota(jnp.int32, (BM, TCOL), 1)
                  + t * TCOL)
            elig = (v > lv) | ((v == lv) & (gi > li))
            vm = jnp.where(elig, v, POS)
            tmin = jnp.min(vm, axis=1, keepdims=True)
            targ = jnp.min(jnp.where(vm == tmin, gi, IBIG),
                           axis=1, keepdims=True)
            take = (tmin < bv) | ((tmin == bv) & (targ < bi))
            return (jnp.where(take, tmin, bv), jnp.where(take, targ, bi))

        bv, bi = lax.fori_loop(
            0, NT, scan_tile,
            (jnp.full((BM, 1), POS, dtype=jnp.float32),
             jnp.full((BM, 1), IBIG, dtype=jnp.int32)))
        idx_ref[:, k:k + 1] = bi
        lv, li = bv, bi


def _run_topk(np_blk, pt_pad):
    return pl.pallas_call(
        _topk_body,
        grid=(MP // BM,),
        in_specs=[
            pl.BlockSpec((BM, 128), lambda i: (i, 0)),
            pl.BlockSpec((8, NP), lambda i: (0, 0)),
        ],
        out_specs=pl.BlockSpec((BM, NSAMPLE), lambda i: (i, 0)),
        out_shape=jax.ShapeDtypeStruct((MP, NSAMPLE), jnp.int32),
        scratch_shapes=[pltpu.VMEM((BM, NP), jnp.float32)],
    )(np_blk, pt_pad)


CH = 4
ROWS_W = MP // 32
NCHUNK = ROWS_W // CH


def _pool_body(xp_hbm, idx_hbm, out_hbm, idx_v, rows_v, pool_v, sem):
    cid = lax.axis_index("c")
    sid = lax.axis_index("s")
    wid = sid * 2 + cid
    base = wid * ROWS_W

    def chunk(ci, _):
        r0 = base + ci * CH
        pltpu.sync_copy(idx_hbm.at[pl.ds(r0 * NSAMPLE, CH * NSAMPLE)], idx_v)
        pltpu.async_copy(xp_hbm.at[idx_v], rows_v, sem).wait()

        def over_r(rr, _):
            def over_s(s, accs):
                row = rr * NSAMPLE + s
                return tuple(
                    jnp.maximum(accs[cc], rows_v[row, pl.ds(cc * 16, 16)])
                    for cc in range(FP // 16))

            accs0 = tuple(rows_v[rr * NSAMPLE, pl.ds(cc * 16, 16)]
                          for cc in range(FP // 16))
            accs = lax.fori_loop(1, NSAMPLE, over_s, accs0)
            prow = ci * CH + rr
            for cc in range(FP // 16):
                pool_v[prow, pl.ds(cc * 16, 16)] = accs[cc]
            return 0

        lax.fori_loop(0, CH, over_r, 0)
        return 0

    lax.fori_loop(0, NCHUNK, chunk, 0)
    pltpu.sync_copy(pool_v, out_hbm.at[pl.ds(base, ROWS_W)])


def _run_pool(xp, idx_flat):
    mesh = plsc.VectorSubcoreMesh(core_axis_name="c", subcore_axis_name="s")
    fn = functools.partial(
        pl.kernel,
        mesh=mesh,
        compiler_params=pltpu.CompilerParams(use_tc_tiling_on_sc=False),
        out_type=jax.ShapeDtypeStruct((MP, FP), jnp.float32),
        scratch_types=[
            pltpu.VMEM((CH * NSAMPLE,), jnp.int32),
            pltpu.VMEM((CH * NSAMPLE, FP), jnp.float32),
            pltpu.VMEM((ROWS_W, FP), jnp.float32),
            pltpu.SemaphoreType.DMA,
        ],
    )(_pool_body)
    return fn(xp, idx_flat)


RT = 256
NRT = MP // RT


def _dense_body(pool_ref, npad_ref, w_ref, sgb_ref, out_ref, h_ref):
    w = w_ref[:, :]
    bvec = sgb_ref[0:1, :]
    gamma = sgb_ref[1:2, :]
    beta = sgb_ref[2:3, :]

    def mm(t, _):
        sl = pl.ds(t * RT, RT)
        adj = pool_ref[sl, :] - npad_ref[sl, :]
        h_ref[sl, :] = jax.lax.dot(adj, w) + bvec
        return 0

    lax.fori_loop(0, NRT, mm, 0)

    def sum_tile(t, acc):
        sl = pl.ds(t * RT, RT)
        gi = lax.broadcasted_iota(jnp.int32, (RT, 1), 0) + t * RT
        msk = gi < M
        return acc + jnp.sum(jnp.where(msk, h_ref[sl, :], 0.0),
                             axis=0, keepdims=True)

    mean = lax.fori_loop(0, NRT, sum_tile,
                         jnp.zeros((1, OUT), jnp.float32)) / M

    def var_tile(t, acc):
        sl = pl.ds(t * RT, RT)
        gi = lax.broadcasted_iota(jnp.int32, (RT, 1), 0) + t * RT
        msk = gi < M
        d = h_ref[sl, :] - mean
        return acc + jnp.sum(jnp.where(msk, d * d, 0.0),
                             axis=0, keepdims=True)

    var = lax.fori_loop(0, NRT, var_tile,
                        jnp.zeros((1, OUT), jnp.float32)) / M
    inv = lax.rsqrt(var + 1e-5)

    def norm_tile(t, _):
        sl = pl.ds(t * RT, RT)
        d = h_ref[sl, :] - mean
        out_ref[sl, :] = jnp.maximum(d * inv * gamma + beta, 0.0)
        return 0

    lax.fori_loop(0, NRT, norm_tile, 0)


def _run_dense(pooled, npad, w_pad, sgb):
    return pl.pallas_call(
        _dense_body,
        out_shape=jax.ShapeDtypeStruct((MP, OUT), jnp.float32),
        scratch_shapes=[pltpu.VMEM((MP, OUT), jnp.float32)],
    )(pooled, npad, w_pad, sgb)


def kernel(p, x, o, W, b, gamma, beta):
    n_p = p[::STRIDE]
    n_o = o // STRIDE

    np_blk = jnp.zeros((MP, 128), jnp.float32).at[:M, :3].set(n_p)
    pt_pad = jnp.full((8, NP), 0.0, jnp.float32)
    pt_pad = pt_pad.at[:3, :N].set(p.T)
    pt_pad = pt_pad.at[:3, N:].set(1e18)
    nn_idx = _run_topk(np_blk, pt_pad)

    xp = jnp.zeros((N, FP), jnp.float32)
    xp = xp.at[:, :3].set(p).at[:, 3:3 + C].set(x)
    pooled = _run_pool(xp, nn_idx.reshape(-1))

    npad = jnp.zeros((MP, FP), jnp.float32).at[:M, :3].set(n_p)
    w_pad = jnp.zeros((FP, OUT), jnp.float32).at[:F, :].set(W)
    sgb = jnp.zeros((8, OUT), jnp.float32)
    sgb = sgb.at[0].set(b).at[1].set(gamma).at[2].set(beta)
    x_out = _run_dense(pooled, npad, w_pad, sgb)[:M]

    return (n_p, x_out, n_o)


# speedup vs baseline: 2.7715x; 1.0928x over previous
import functools

import jax
import jax.numpy as jnp
from jax import lax
from jax.experimental import pallas as pl
from jax.experimental.pallas import tpu as pltpu
from jax.experimental.pallas import tpu_sc as plsc

N = 10000
C = 128
OUT = 256
STRIDE = 4
NSAMPLE = 32

M = N // STRIDE
MP = 2560
NP = 10240
BM = 512
TCOL = 1024
NT = NP // TCOL
F = C + 3
FP = 144

NEG = -3.0e38
POS = 3.0e38
IBIG = 2 ** 30


def _topk_body(np_ref, pt_ref, idx_ref, cur_ref):
    a0 = np_ref[:, 0:1]
    a1 = np_ref[:, 1:2]
    a2 = np_ref[:, 2:3]
    pn = a0 * a0 + a1 * a1 + a2 * a2

    def fill(t, _):
        sl = pl.ds(t * TCOL, TCOL)
        p0 = pt_ref[0:1, sl]
        p1 = pt_ref[1:2, sl]
        p2 = pt_ref[2:3, sl]
        pp = p0 * p0 + p1 * p1 + p2 * p2
        mm = jax.lax.dot_general(
            np_ref[:, 0:3], pt_ref[0:3, sl],
            (((1,), (0,)), ((), ())))
        cur_ref[:, sl] = pn + pp - 2.0 * mm
        return 0

    lax.fori_loop(0, NT, fill, 0)

    lv = jnp.full((BM, 1), NEG, dtype=jnp.float32)
    li = jnp.full((BM, 1), -1, dtype=jnp.int32)
    for k in range(NSAMPLE):
        def scan_tile(t, carry):
            bv, bi = carry
            sl = pl.ds(t * TCOL, TCOL)
            v = cur_ref[:, sl]
            gi = (lax.broadcasted_iota(jnp.int32, (BM, TCOL), 1)
                  + t * TCOL)
            elig = (v > lv) | ((v == lv) & (gi > li))
            vm = jnp.where(elig, v, POS)
            tmin = jnp.min(vm, axis=1, keepdims=True)
            targ = jnp.min(jnp.where(vm == tmin, gi, IBIG),
                           axis=1, keepdims=True)
            take = (tmin < bv) | ((tmin == bv) & (targ < bi))
            return (jnp.where(take, tmin, bv), jnp.where(take, targ, bi))

        bv, bi = lax.fori_loop(
            0, NT, scan_tile,
            (jnp.full((BM, 1), POS, dtype=jnp.float32),
             jnp.full((BM, 1), IBIG, dtype=jnp.int32)))
        idx_ref[:, k:k + 1] = bi
        lv, li = bv, bi


def _run_topk(np_blk, pt_pad):
    return pl.pallas_call(
        _topk_body,
        grid=(MP // BM,),
        in_specs=[
            pl.BlockSpec((BM, 128), lambda i: (i, 0)),
            pl.BlockSpec((8, NP), lambda i: (0, 0)),
        ],
        out_specs=pl.BlockSpec((BM, NSAMPLE), lambda i: (i, 0)),
        out_shape=jax.ShapeDtypeStruct((MP, NSAMPLE), jnp.int32),
        scratch_shapes=[pltpu.VMEM((BM, NP), jnp.float32)],
    )(np_blk, pt_pad)


CH = 4
ROWS_W = MP // 32
NCHUNK = ROWS_W // CH


def _pool_body(xp_hbm, idx_hbm, out_hbm, idx_v, rows_v, pool_v, sem):
    cid = lax.axis_index("c")
    sid = lax.axis_index("s")
    wid = sid * 2 + cid
    base = wid * ROWS_W

    def chunk(ci, _):
        r0 = base + ci * CH
        pltpu.sync_copy(idx_hbm.at[pl.ds(r0 * NSAMPLE, CH * NSAMPLE)], idx_v)
        pltpu.async_copy(xp_hbm.at[idx_v], rows_v, sem).wait()

        def over_r(rr, _):
            def over_s(s, accs):
                row = rr * NSAMPLE + s
                return tuple(
                    jnp.maximum(accs[cc], rows_v[row, pl.ds(cc * 16, 16)])
                    for cc in range(FP // 16))

            accs0 = tuple(rows_v[rr * NSAMPLE, pl.ds(cc * 16, 16)]
                          for cc in range(FP // 16))
            accs = lax.fori_loop(1, NSAMPLE, over_s, accs0)
            prow = ci * CH + rr
            for cc in range(FP // 16):
                pool_v[prow, pl.ds(cc * 16, 16)] = accs[cc]
            return 0

        lax.fori_loop(0, CH, over_r, 0)
        return 0

    lax.fori_loop(0, NCHUNK, chunk, 0)
    pltpu.sync_copy(pool_v, out_hbm.at[pl.ds(base, ROWS_W)])


def _run_pool(xp, idx_flat):
    mesh = plsc.VectorSubcoreMesh(core_axis_name="c", subcore_axis_name="s")
    fn = functools.partial(
        pl.kernel,
        mesh=mesh,
        compiler_params=pltpu.CompilerParams(use_tc_tiling_on_sc=False),
        out_type=jax.ShapeDtypeStruct((MP, FP), jnp.float32),
        scratch_types=[
            pltpu.VMEM((CH * NSAMPLE,), jnp.int32),
            pltpu.VMEM((CH * NSAMPLE, FP), jnp.float32),
            pltpu.VMEM((ROWS_W, FP), jnp.float32),
            pltpu.SemaphoreType.DMA,
        ],
    )(_pool_body)
    return fn(xp, idx_flat)


RT = 256
NRT = MP // RT


def _dense_body(pool_ref, npad_ref, w_ref, sgb_ref, out_ref, h_ref):
    w = w_ref[:, :]
    bvec = sgb_ref[0:1, :]
    gamma = sgb_ref[1:2, :]
    beta = sgb_ref[2:3, :]

    def mm(t, _):
        sl = pl.ds(t * RT, RT)
        adj = pool_ref[sl, :] - npad_ref[sl, :]
        h_ref[sl, :] = jax.lax.dot(adj, w) + bvec
        return 0

    lax.fori_loop(0, NRT, mm, 0)

    def sum_tile(t, acc):
        sl = pl.ds(t * RT, RT)
        gi = lax.broadcasted_iota(jnp.int32, (RT, 1), 0) + t * RT
        msk = gi < M
        return acc + jnp.sum(jnp.where(msk, h_ref[sl, :], 0.0),
                             axis=0, keepdims=True)

    mean = lax.fori_loop(0, NRT, sum_tile,
                         jnp.zeros((1, OUT), jnp.float32)) / M

    def var_tile(t, acc):
        sl = pl.ds(t * RT, RT)
        gi = lax.broadcasted_iota(jnp.int32, (RT, 1), 0) + t * RT
        msk = gi < M
        d = h_ref[sl, :] - mean
        return acc + jnp.sum(jnp.where(msk, d * d, 0.0),
                             axis=0, keepdims=True)

    var = lax.fori_loop(0, NRT, var_tile,
                        jnp.zeros((1, OUT), jnp.float32)) / M
    inv = lax.rsqrt(var + 1e-5)

    def norm_tile(t, _):
        sl = pl.ds(t * RT, RT)
        d = h_ref[sl, :] - mean
        out_ref[sl, :] = jnp.maximum(d * inv * gamma + beta, 0.0)
        return 0

    lax.fori_loop(0, NRT, norm_tile, 0)


def _run_dense(pooled, npad, w_pad, sgb):
    return pl.pallas_call(
        _dense_body,
        out_shape=jax.ShapeDtypeStruct((MP, OUT), jnp.float32),
        scratch_shapes=[pltpu.VMEM((MP, OUT), jnp.float32)],
    )(pooled, npad, w_pad, sgb)


def kernel(p, x, o, W, b, gamma, beta):
    n_p = p[::STRIDE]
    n_o = o // STRIDE

    np_blk = jnp.zeros((MP, 128), jnp.float32).at[:M, :3].set(n_p)
    pt_pad = jnp.full((8, NP), 0.0, jnp.float32)
    pt_pad = pt_pad.at[:3, :N].set(p.T)
    pt_pad = pt_pad.at[:3, N:].set(1e18)
    nn_idx = _run_topk(np_blk, pt_pad)

    xp = jnp.zeros((N, FP), jnp.float32)
    xp = xp.at[:, :3].set(p).at[:, 3:3 + C].set(x)
    pooled = _run_pool(xp, nn_idx.reshape(-1))

    npad = jnp.zeros((MP, FP), jnp.float32).at[:M, :3].set(n_p)
    w_pad = jnp.zeros((FP, OUT), jnp.float32).at[:F, :].set(W)
    sgb = jnp.zeros((8, OUT), jnp.float32)
    sgb = sgb.at[0].set(b).at[1].set(gamma).at[2].set(beta)
    x_out = _run_dense(pooled, npad, w_pad, sgb)[:M]

    return (n_p, x_out, n_o)
